# Initial kernel scaffold; baseline (speedup 1.0000x reference)
import jax
import jax.numpy as jnp
from jax.experimental import pallas as pl


def kernel(sent_a, table, bias):
    raise NotImplementedError("write your pallas kernel here")


# speedup vs baseline: 10.3946x; 10.3946x over previous
import functools

import jax
import jax.numpy as jnp
from jax import lax
from jax.experimental import pallas as pl
from jax.experimental.pallas import tpu as pltpu
from jax.experimental.pallas import tpu_sc as plsc

_D = 128
_LANES = 16
_ND = _D // _LANES
_NC = 2
_NS = 16
_NW = _NC * _NS
_G = 2


@functools.lru_cache(maxsize=None)
def _build(B, H, V):
    b_per_w = B // _NW
    ngroups = b_per_w // _G
    k = _G * H

    mesh = plsc.VectorSubcoreMesh(core_axis_name="c", subcore_axis_name="s")

    @functools.partial(
        pl.kernel,
        out_type=jax.ShapeDtypeStruct((B, _D), jnp.float32),
        mesh=mesh,
        scratch_types=[
            pltpu.VMEM((ngroups, k), jnp.int32),
            pltpu.VMEM((2, k, _D), jnp.float32),
            pltpu.VMEM((b_per_w, _D), jnp.float32),
            pltpu.VMEM((_D,), jnp.float32),
            pltpu.SemaphoreType.DMA,
            pltpu.SemaphoreType.DMA,
            pltpu.SemaphoreType.DMA,
        ],
    )
    def emb_sum(idx_hbm, table_hbm, bias_hbm, out_hbm,
                idx_v, buf_v, out_v, bias_v, sem0, sem1, sem_io):
        wid = lax.axis_index("s") * _NC + lax.axis_index("c")
        pltpu.async_copy(bias_hbm, bias_v, sem_io).wait()
        pltpu.async_copy(idx_hbm.at[wid], idx_v, sem_io).wait()

        sems = (sem0, sem1)
        pltpu.async_copy(table_hbm.at[idx_v.at[0]], buf_v.at[0], sems[0])
        for g in range(ngroups):
            slot = g % 2
            if g + 1 < ngroups:
                nslot = (g + 1) % 2
                pltpu.async_copy(
                    table_hbm.at[idx_v.at[g + 1]], buf_v.at[nslot], sems[nslot])
            pltpu.make_async_copy(
                table_hbm.at[idx_v.at[g]], buf_v.at[slot], sems[slot]).wait()
            for r in range(_G):
                accs = tuple(
                    bias_v[pl.ds(d * _LANES, _LANES)] for d in range(_ND))

                def body(l, accs, _slot=slot, _r=r):
                    return tuple(
                        accs[d]
                        + buf_v[_slot, _r * H + l, pl.ds(d * _LANES, _LANES)]
                        for d in range(_ND))

                accs = lax.fori_loop(0, H, body, accs)
                for d in range(_ND):
                    out_v[g * _G + r, pl.ds(d * _LANES, _LANES)] = accs[d]

        pltpu.async_copy(
            out_v, out_hbm.at[pl.ds(wid * b_per_w, b_per_w)], sem_io).wait()

    return emb_sum


def kernel(sent_a, table, bias):
    B, H = sent_a.shape
    V, D = table.shape
    assert D == _D and B % (_NW * _G) == 0
    idx = sent_a.astype(jnp.int32).reshape(_NW, (B // _NW) // _G, _G * H)
    return _build(B, H, V)(idx, table, bias)


# speedup vs baseline: 15.1982x; 1.4621x over previous
import functools

import jax
import jax.numpy as jnp
from jax import lax
from jax.experimental import pallas as pl
from jax.experimental.pallas import tpu as pltpu
from jax.experimental.pallas import tpu_sc as plsc

_D = 128
_LANES = 16
_ND = _D // _LANES
_NC = 2
_NS = 16
_NW = _NC * _NS
_G = 2
_NBUF = 4
_UNROLL = 5


@functools.lru_cache(maxsize=None)
def _build(B, H, V):
    b_per_w = B // _NW
    ngroups = b_per_w // _G
    k = _G * H
    assert H % _UNROLL == 0 and ngroups % _NBUF == 0

    mesh = plsc.VectorSubcoreMesh(core_axis_name="c", subcore_axis_name="s")

    @functools.partial(
        pl.kernel,
        out_type=jax.ShapeDtypeStruct((B, _D), jnp.float32),
        mesh=mesh,
        scratch_types=[
            pltpu.VMEM((ngroups, k), jnp.int32),
            pltpu.VMEM((_NBUF, k, _D), jnp.float32),
            pltpu.VMEM((b_per_w, _D), jnp.float32),
            pltpu.VMEM((_D,), jnp.float32),
            pltpu.SemaphoreType.DMA,
            pltpu.SemaphoreType.DMA,
            pltpu.SemaphoreType.DMA,
            pltpu.SemaphoreType.DMA,
            pltpu.SemaphoreType.DMA,
        ],
    )
    def emb_sum(idx_hbm, table_hbm, bias_hbm, out_hbm,
                idx_v, buf_v, out_v, bias_v, sem0, sem1, sem2, sem3, sem_io):
        wid = lax.axis_index("s") * _NC + lax.axis_index("c")
        pltpu.async_copy(bias_hbm, bias_v, sem_io).wait()
        pltpu.async_copy(idx_hbm.at[wid], idx_v, sem_io).wait()

        sems = (sem0, sem1, sem2, sem3)
        for slot in range(_NBUF - 1):
            pltpu.async_copy(
                table_hbm.at[idx_v.at[slot]], buf_v.at[slot], sems[slot])

        def ring(j, _):
            for b in range(_NBUF):
                g = _NBUF * j + b
                pltpu.make_async_copy(
                    table_hbm.at[idx_v.at[g]], buf_v.at[b], sems[b]).wait()

                nslot = (b + _NBUF - 1) % _NBUF

                @pl.when(g + _NBUF - 1 < ngroups)
                def _prefetch(_g=g, _ns=nslot):
                    pltpu.async_copy(
                        table_hbm.at[idx_v.at[_g + _NBUF - 1]],
                        buf_v.at[_ns], sems[_ns])

                for r in range(_G):
                    accs = tuple(
                        bias_v[pl.ds(d * _LANES, _LANES)]
                        for d in range(_ND))

                    def body(l, accs, _b=b, _r=r):
                        for u in range(_UNROLL):
                            accs = tuple(
                                accs[d] + buf_v[_b,
                                                _r * H + l * _UNROLL + u,
                                                pl.ds(d * _LANES, _LANES)]
                                for d in range(_ND))
                        return accs

                    accs = lax.fori_loop(0, H // _UNROLL, body, accs)
                    for d in range(_ND):
                        out_v[g * _G + r, pl.ds(d * _LANES, _LANES)] = accs[d]
            return ()

        lax.fori_loop(0, ngroups // _NBUF, ring, ())
        pltpu.async_copy(
            out_v, out_hbm.at[pl.ds(wid * b_per_w, b_per_w)], sem_io).wait()

    return emb_sum


def kernel(sent_a, table, bias):
    B, H = sent_a.shape
    V, D = table.shape
    assert D == _D and B % (_NW * _G) == 0
    idx = sent_a.astype(jnp.int32).reshape(_NW, (B // _NW) // _G, _G * H)
    return _build(B, H, V)(idx, table, bias)


# speedup vs baseline: 15.2290x; 1.0020x over previous
import functools

import jax
import jax.numpy as jnp
from jax import lax
from jax.experimental import pallas as pl
from jax.experimental.pallas import tpu as pltpu
from jax.experimental.pallas import tpu_sc as plsc

_D = 128
_LANES = 16
_ND = _D // _LANES
_NC = 2
_NS = 16
_NW = _NC * _NS
_G = 2
_NBUF = 4
_UNROLL = 5


@functools.lru_cache(maxsize=None)
def _build(B, H, V):
    b_per_w = B // _NW
    ngroups = b_per_w // _G
    k = _G * H
    assert H % _UNROLL == 0 and ngroups % _NBUF == 0

    mesh = plsc.VectorSubcoreMesh(core_axis_name="c", subcore_axis_name="s")

    @functools.partial(
        pl.kernel,
        out_type=jax.ShapeDtypeStruct((B, _D), jnp.float32),
        mesh=mesh,
        scratch_types=[
            pltpu.VMEM((ngroups, k), jnp.int32),
            pltpu.VMEM((_NBUF, k, _D), jnp.float32),
            pltpu.VMEM((b_per_w, _D), jnp.float32),
            pltpu.VMEM((_D,), jnp.float32),
            pltpu.SemaphoreType.DMA,
            pltpu.SemaphoreType.DMA,
            pltpu.SemaphoreType.DMA,
            pltpu.SemaphoreType.DMA,
            pltpu.SemaphoreType.DMA,
        ],
    )
    def emb_sum(idx_hbm, table_hbm, bias_hbm, out_hbm,
                idx_v, buf_v, out_v, bias_v, sem0, sem1, sem2, sem3, sem_io):
        wid = lax.axis_index("s") * _NC + lax.axis_index("c")
        pltpu.async_copy(bias_hbm, bias_v, sem_io).wait()
        pltpu.async_copy(idx_hbm.at[wid], idx_v, sem_io).wait()

        sems = (sem0, sem1, sem2, sem3)
        for slot in range(_NBUF - 1):
            pltpu.async_copy(
                table_hbm.at[idx_v.at[slot]], buf_v.at[slot], sems[slot])

        bias_regs = tuple(
            bias_v[pl.ds(d * _LANES, _LANES)] for d in range(_ND))

        def ring(j, bias_regs):
            for b in range(_NBUF):
                g = _NBUF * j + b
                pltpu.make_async_copy(
                    table_hbm.at[idx_v.at[g]], buf_v.at[b], sems[b]).wait()

                nslot = (b + _NBUF - 1) % _NBUF

                @pl.when(g + _NBUF - 1 < ngroups)
                def _prefetch(_g=g, _ns=nslot):
                    pltpu.async_copy(
                        table_hbm.at[idx_v.at[_g + _NBUF - 1]],
                        buf_v.at[_ns], sems[_ns])

                accs = bias_regs + bias_regs

                def body(l, accs, _b=b):
                    for u in range(_UNROLL):
                        accs = tuple(
                            accs[r * _ND + d]
                            + buf_v[_b, r * H + l * _UNROLL + u,
                                    pl.ds(d * _LANES, _LANES)]
                            for r in range(_G) for d in range(_ND))
                    return accs

                accs = lax.fori_loop(0, H // _UNROLL, body, accs)
                for r in range(_G):
                    for d in range(_ND):
                        out_v[g * _G + r, pl.ds(d * _LANES, _LANES)] = (
                            accs[r * _ND + d])
            return bias_regs

        lax.fori_loop(0, ngroups // _NBUF, ring, bias_regs)
        pltpu.async_copy(
            out_v, out_hbm.at[pl.ds(wid * b_per_w, b_per_w)], sem_io).wait()

    return emb_sum


def kernel(sent_a, table, bias):
    B, H = sent_a.shape
    V, D = table.shape
    assert D == _D and B % (_NW * _G) == 0
    idx = sent_a.astype(jnp.int32).reshape(_NW, (B // _NW) // _G, _G * H)
    return _build(B, H, V)(idx, table, bias)
